# baseline (speedup 1.0000x reference)
import functools
import math

import jax
import jax.numpy as jnp
from jax import lax
from jax.experimental import pallas as pl
from jax.experimental.pallas import tpu as pltpu
from jax.experimental.pallas import tpu_sc as plsc

_N = 1024
_SIZE = 100000
_PAD = 0
_SMOOTH = 0.1
_CONF = 1.0 - _SMOOTH
_EPS = _SMOOTH / (_SIZE - 2)
_CCONST = (_SIZE - 2) * _EPS * math.log(_EPS) + _CONF * math.log(_CONF)

_NTILES = 32
_RPT = _N // _NTILES

_NTC = 512
_NSC = _N - _NTC
_RSC = _NSC // _NTILES

_GR = 8
_NGRP = None
_CCOL = 2560
_NCH = _SIZE // _CCOL
_TAIL0 = _NCH * _CCOL
_TAIL1 = _TAIL0 + 128

_BR = 16
_NSLAB = _NTC // _BR
_NBUF = 8


def _slab_copy(x_any, bufs, sems, slab, slot):
    return pltpu.make_async_copy(
        x_any.at[pl.ds(slab * _BR, _BR), :],
        bufs.at[slot],
        sems.at[slot],
    )


def _tc_body(x_any, t_ref, out_ref, bufs, sems, acc_ref):
    i = pl.program_id(0)

    @pl.when(i == 0)
    def _():
        acc_ref[0, 0] = jnp.float32(0.0)
        for b in range(_NBUF):
            _slab_copy(x_any, bufs, sems, b, b).start()

    slot = lax.rem(i, _NBUF)
    _slab_copy(x_any, bufs, sems, i, slot).wait()

    xb = bufs[slot]
    rowsum = jnp.sum(xb, axis=1, keepdims=True)
    per_row = _CCONST + _EPS * (xb[:, 0:1] - rowsum)
    valid = t_ref[pl.ds(i * _BR, _BR), :] != _PAD
    part = jnp.sum(jnp.where(valid, per_row, 0.0))
    acc_ref[0, 0] += part

    @pl.when(i + _NBUF < _NSLAB)
    def _():
        _slab_copy(x_any, bufs, sems, i + _NBUF, slot).start()

    @pl.when(i == _NSLAB - 1)
    def _():
        out_ref[...] = jnp.broadcast_to(acc_ref[0, 0], (1, 1))


_tc_call = pl.pallas_call(
    _tc_body,
    grid=(_NSLAB,),
    in_specs=[
        pl.BlockSpec(memory_space=pl.MemorySpace.ANY),
        pl.BlockSpec((_N, 1), lambda i: (0, 0)),
    ],
    out_specs=pl.BlockSpec((1, 1), lambda i: (0, 0)),
    out_shape=jax.ShapeDtypeStruct((1, 1), jnp.float32),
    scratch_shapes=[
        pltpu.VMEM((_NBUF, _BR, _SIZE), jnp.float32),
        pltpu.SemaphoreType.DMA((_NBUF,)),
        pltpu.SMEM((1, 1), jnp.float32),
    ],
    compiler_params=pltpu.CompilerParams(
        dimension_semantics=("arbitrary",),
    ),
)


def _lane0_f32():
    iota = lax.broadcasted_iota(jnp.int32, (16,), 0)
    return jnp.maximum(1 - jnp.abs(iota), 0).astype(jnp.float32)


def _sc_body(x_hbm, t_hbm, out_hbm, tv, tv2, rowbuf, accbuf, cbufs, tbuf0,
             tbuf1, csems):
    c = lax.axis_index("c")
    s = lax.axis_index("s")
    wid = s * 2 + c
    iota = lax.broadcasted_iota(jnp.int32, (16,), 0)

    sbase = _NTC + wid * _RSC
    pltpu.sync_copy(t_hbm.at[pl.ds(sbase, _RSC)], tv2)
    tvec2 = tv2[pl.ds(0, 16)]

    total = jnp.zeros((16,), jnp.float32)
    misc = jnp.float32(0.0)
    ngrp = _RSC // _GR
    nchunks = ngrp * _NCH

    def _chunk_copy(m):
        g = m // _NCH
        ci = m % _NCH
        return pltpu.make_async_copy(
            x_hbm.at[pl.ds(sbase + g * _GR, _GR), pl.ds(ci * _CCOL, _CCOL)],
            cbufs.at[m % 2],
            csems.at[m % 2],
        )

    def _wrow(g, r):
        t = tvec2[g * _GR + r]
        return jnp.minimum(jnp.abs(t), 1).astype(jnp.float32)

    _chunk_copy(0).start()
    for m in range(nchunks):
        g = m // _NCH
        ci = m % _NCH
        par = m % 2
        if m + 1 < nchunks:
            _chunk_copy(m + 1).start()
        _chunk_copy(m).wait()
        w = [_wrow(g, r) for r in range(_GR)]

        @plsc.parallel_loop(0, _CCOL, 16, unroll=2,
                            carry=jnp.zeros((16,), jnp.float32))
        def _ch_acc(j, a):
            v = [cbufs[par, r, pl.ds(j, 16)] * w[r] for r in range(_GR)]
            s01 = (v[0] + v[1]) + (v[2] + v[3])
            s23 = (v[4] + v[5]) + (v[6] + v[7])
            return a + (s01 + s23)

        total = total + _ch_acc * (-_EPS)
        if ci == 0:
            for r in range(_GR):
                x0 = cbufs[par, r, pl.ds(0, 16)][0]
                misc = misc + w[r] * (_CCONST + _EPS * x0)
        if ci == _NCH - 1:
            pltpu.sync_copy(
                x_hbm.at[pl.ds(sbase + g * _GR, _GR), pl.ds(_TAIL0, 128)],
                tbuf0)
            pltpu.sync_copy(
                x_hbm.at[pl.ds(sbase + g * _GR, _GR), pl.ds(_TAIL1, 32)],
                tbuf1)
            tail = jnp.zeros((16,), jnp.float32)
            for r in range(_GR):
                for j in range(8):
                    tail = tail + tbuf0[r, pl.ds(j * 16, 16)] * w[r]
                for j in range(2):
                    tail = tail + tbuf1[r, pl.ds(j * 16, 16)] * w[r]
            total = total + tail * (-_EPS)

    total = total + misc * _lane0_f32()

    gbase = wid * _RPT
    pltpu.sync_copy(t_hbm.at[pl.ds(gbase, _RPT)], tv)
    gacc = jnp.zeros((16,), jnp.float32)
    for k in range(_RPT):
        t = tv[pl.ds((k // 16) * 16, 16)][k % 16]
        off = (t // 16) * 16
        pltpu.sync_copy(x_hbm.at[gbase + k, pl.ds(off, 16)], rowbuf)
        valid = jnp.minimum(jnp.abs(t), 1)
        ind = jnp.maximum(1 - jnp.abs(iota - (t - off)), 0) * valid
        gacc = gacc + rowbuf[...] * ind.astype(jnp.float32)

    accbuf[...] = total + gacc * (_EPS - _CONF)
    pltpu.sync_copy(accbuf, out_hbm.at[pl.ds(wid * 16, 16)])


@functools.cache
def _get_sc_call():
    return functools.partial(
        pl.kernel,
        out_type=jax.ShapeDtypeStruct((_NTILES * 16,), jnp.float32),
        mesh=plsc.VectorSubcoreMesh(core_axis_name="c", subcore_axis_name="s"),
        scratch_types=[
            pltpu.VMEM((_RPT,), jnp.int32),
            pltpu.VMEM((_RSC,), jnp.int32),
            pltpu.VMEM((16,), jnp.float32),
            pltpu.VMEM((16,), jnp.float32),
            pltpu.VMEM((2, _GR, _CCOL), jnp.float32),
            pltpu.VMEM((_GR, 128), jnp.float32),
            pltpu.VMEM((_GR, 32), jnp.float32),
            pltpu.SemaphoreType.DMA((2,)),
        ],
    )(_sc_body)


def kernel(x, target):
    target = target.astype(jnp.int32)
    tc_out = _tc_call(x, target.reshape(_N, 1))
    sc_out = _get_sc_call()(x, target)
    return tc_out[0, 0] + jnp.sum(sc_out)
